# Initial kernel scaffold; baseline (speedup 1.0000x reference)
import functools

import jax
import jax.numpy as jnp
from jax import lax
from jax.experimental import pallas as pl
from jax.experimental.pallas import tpu as pltpu
from jax.experimental.pallas import tpu_sc as plsc

NC, NS, L = 2, 16, 16
NW = NC * NS


def _match_body(B, FLAT, PPW, perms_hbm, buf_hbm, scores_hbm, idx_out_hbm,
                cand_f, cand_i, row_f, scores_v, idxrow_v):
    wid = lax.axis_index("s") * NC + lax.axis_index("c")
    p0 = wid * PPW
    pltpu.sync_copy(perms_hbm.at[pl.ds(p0, PPW)], cand_f)
    pltpu.sync_copy(scores_hbm, scores_v)
    nk = FLAT // L
    for c in range(PPW):
        for k in range(nk):
            cand_i[c, pl.ds(k * L, L)] = cand_f[c, pl.ds(k * L, L)].astype(jnp.int32)

    def cond(carry):
        b = carry[0]
        found_all = carry[1] >= 0
        for c in range(1, PPW):
            found_all = found_all & (carry[1 + c] >= 0)
        return (b < B) & jnp.logical_not(found_all)

    def body(carry):
        b = carry[0]
        pltpu.sync_copy(buf_hbm.at[b], row_f)
        accs = [jnp.zeros((L,), jnp.int32) for _ in range(PPW)]
        for k in range(nk):
            r = row_f[pl.ds(k * L, L)].astype(jnp.int32)
            for c in range(PPW):
                accs[c] = accs[c] | (r ^ cand_i[c, pl.ds(k * L, L)])
        new_idx = []
        for c in range(PPW):
            match = jnp.max(accs[c]) == 0
            prev = carry[1 + c]
            new_idx.append(jnp.where((prev < 0) & match, b, prev))
        return (b + 1, *new_idx)

    init = (jnp.int32(0),) + tuple(jnp.int32(-1) for _ in range(PPW))
    final = lax.while_loop(cond, body, init)

    lanes = lax.iota(jnp.int32, L)
    iv = jnp.full((L,), -1, jnp.int32)
    for c in range(PPW):
        iv = jnp.where(lanes == c, final[1 + c], iv)
    g = plsc.load_gather(scores_v, [jnp.maximum(iv, 0)])
    iv = jnp.where((iv >= 0) & (g != -jnp.inf), iv, -1)
    idxrow_v[...] = iv
    pltpu.sync_copy(idxrow_v, idx_out_hbm.at[wid])


def _score_body(B, P, PPW, scores_hbm, new_hbm, idxpad_hbm, out_hbm,
                idx_v, ns_v, bs_v):
    wid = lax.axis_index("s") * NC + lax.axis_index("c")
    rows = B // NW
    base = wid * rows
    pltpu.sync_copy(idxpad_hbm, idx_v)
    pltpu.sync_copy(new_hbm, ns_v)
    pltpu.sync_copy(scores_hbm.at[pl.ds(base, rows)], bs_v)
    lanes = base + lax.iota(jnp.int32, L)
    acc = bs_v[...]
    for p in range(P):
        ip = idx_v[p // PPW, p % PPW]
        sp = ns_v[p]
        acc = jnp.where(lanes == ip, jnp.maximum(acc, sp), acc)
    bs_v[...] = acc
    pltpu.sync_copy(bs_v, out_hbm.at[pl.ds(base, rows)])


def kernel(perm_buffer, buffer_scores, permutations, new_scores):
    B = perm_buffer.shape[0]
    P = permutations.shape[0]
    FLAT = perm_buffer.shape[1] * perm_buffer.shape[2]
    PPW = P // NW

    buf2d = perm_buffer.reshape(B, FLAT)
    perm2d = permutations.reshape(P, FLAT)

    mesh = plsc.VectorSubcoreMesh(core_axis_name="c", subcore_axis_name="s")

    match = functools.partial(
        pl.kernel,
        out_type=jax.ShapeDtypeStruct((NW, L), jnp.int32),
        mesh=mesh,
        scratch_types=[
            pltpu.VMEM((PPW, FLAT), jnp.float32),
            pltpu.VMEM((PPW, FLAT), jnp.int32),
            pltpu.VMEM((FLAT,), jnp.float32),
            pltpu.VMEM((B,), jnp.float32),
            pltpu.VMEM((L,), jnp.int32),
        ],
    )(functools.partial(_match_body, B, FLAT, PPW))
    idx_pad = match(perm2d, buf2d, buffer_scores)

    score = functools.partial(
        pl.kernel,
        out_type=jax.ShapeDtypeStruct((B,), jnp.float32),
        mesh=mesh,
        scratch_types=[
            pltpu.VMEM((NW, L), jnp.int32),
            pltpu.VMEM((P,), jnp.float32),
            pltpu.VMEM((B // NW,), jnp.float32),
        ],
    )(functools.partial(_score_body, B, P, PPW))
    updated = score(buffer_scores, new_scores, idx_pad)

    idx = idx_pad[:, :PPW].reshape(P)
    return (idx, updated)


# speedup vs baseline: 2.6164x; 2.6164x over previous
import functools

import jax
import jax.numpy as jnp
from jax import lax
from jax.experimental import pallas as pl
from jax.experimental.pallas import tpu as pltpu
from jax.experimental.pallas import tpu_sc as plsc

NC, NS, L = 2, 16, 16
NW = NC * NS


def _match_body(B, FLAT, PPW, CH, perms_hbm, buf_hbm, scores_hbm, idx_out_hbm,
                cand_f, cand_i, rows_v, gath_v, idxrow_v, sem):
    wid = lax.axis_index("s") * NC + lax.axis_index("c")
    p0 = wid * PPW
    pltpu.sync_copy(perms_hbm.at[pl.ds(p0, PPW)], cand_f)
    nk = FLAT // L
    for c in range(PPW):
        for k in range(nk):
            cand_i[c, pl.ds(k * L, L)] = cand_f[c, pl.ds(k * L, L)].astype(jnp.int32)

    lanes = lax.iota(jnp.int32, L)

    def lane_or(x):
        for sh in (8, 4, 2, 1):
            x = x | x[lanes ^ sh]
        return x

    def all_found(iv):
        pending = jnp.where((iv < 0) & (lanes < PPW), 1, 0)
        return lane_or(pending)[0] == 0

    idxrow_v[...] = jnp.full((L,), -1, jnp.int32)

    def chunk_body(chunk, dummy):
        @pl.when(jnp.logical_not(all_found(idxrow_v[...])))
        def _():
            pltpu.async_copy(buf_hbm.at[pl.ds(chunk * CH, CH)], rows_v, sem).wait()

            def row_body(rb, d2):
                iv = idxrow_v[...]

                @pl.when(jnp.logical_not(all_found(iv)))
                def _():
                    accs = [jnp.zeros((L,), jnp.int32) for _ in range(PPW)]
                    for k in range(nk):
                        r = rows_v[rb, pl.ds(k * L, L)].astype(jnp.int32)
                        for c in range(PPW):
                            accs[c] = accs[c] | (r ^ cand_i[c, pl.ds(k * L, L)])
                    b = chunk * CH + rb
                    iv2 = iv
                    for c in range(PPW):
                        match_vec = lane_or(accs[c]) == 0
                        iv2 = jnp.where((lanes == c) & (iv2 < 0) & match_vec, b, iv2)
                    idxrow_v[...] = iv2

                return d2

            lax.fori_loop(0, CH, row_body, jnp.int32(0))

        return dummy

    lax.fori_loop(0, B // CH, chunk_body, jnp.int32(0))

    iv = idxrow_v[...]
    pltpu.async_copy(scores_hbm.at[jnp.maximum(iv, 0)], gath_v, sem).wait()
    g = gath_v[...]
    iv = jnp.where((iv >= 0) & (g != -jnp.inf), iv, -1)
    idxrow_v[...] = iv
    pltpu.sync_copy(idxrow_v, idx_out_hbm.at[wid])


def _score_body(B, P, PPW, scores_hbm, new_hbm, idxpad_hbm, out_hbm,
                idx_v, ns_v, bs_v):
    wid = lax.axis_index("s") * NC + lax.axis_index("c")
    rows = B // NW
    base = wid * rows
    pltpu.sync_copy(idxpad_hbm, idx_v)
    pltpu.sync_copy(new_hbm, ns_v)
    pltpu.sync_copy(scores_hbm.at[pl.ds(base, rows)], bs_v)
    lanes = base + lax.iota(jnp.int32, L)
    acc = bs_v[...]
    for t in range(P // L):
        nsvec = ns_v[pl.ds(t * L, L)]
        rowvecs = [idx_v[t * (L // PPW) + r] for r in range(L // PPW)]
        for j in range(L):
            ip = rowvecs[j // PPW][j % PPW]
            sp = nsvec[j]
            acc = jnp.where(lanes == ip, jnp.maximum(acc, sp), acc)
    bs_v[...] = acc
    pltpu.sync_copy(bs_v, out_hbm.at[pl.ds(base, rows)])


def kernel(perm_buffer, buffer_scores, permutations, new_scores):
    B = perm_buffer.shape[0]
    P = permutations.shape[0]
    FLAT = perm_buffer.shape[1] * perm_buffer.shape[2]
    PPW = P // NW

    buf2d = perm_buffer.reshape(B, FLAT)
    perm2d = permutations.reshape(P, FLAT)

    CH = 8
    mesh = plsc.VectorSubcoreMesh(core_axis_name="c", subcore_axis_name="s")

    match = functools.partial(
        pl.kernel,
        out_type=jax.ShapeDtypeStruct((NW, L), jnp.int32),
        mesh=mesh,
        scratch_types=[
            pltpu.VMEM((PPW, FLAT), jnp.float32),
            pltpu.VMEM((PPW, FLAT), jnp.int32),
            pltpu.VMEM((CH, FLAT), jnp.float32),
            pltpu.VMEM((L,), jnp.float32),
            pltpu.VMEM((L,), jnp.int32),
            pltpu.SemaphoreType.DMA,
        ],
    )(functools.partial(_match_body, B, FLAT, PPW, CH))
    idx_pad = match(perm2d, buf2d, buffer_scores)

    score = functools.partial(
        pl.kernel,
        out_type=jax.ShapeDtypeStruct((B,), jnp.float32),
        mesh=mesh,
        scratch_types=[
            pltpu.VMEM((NW, L), jnp.int32),
            pltpu.VMEM((P,), jnp.float32),
            pltpu.VMEM((B // NW,), jnp.float32),
        ],
    )(functools.partial(_score_body, B, P, PPW))
    updated = score(buffer_scores, new_scores, idx_pad)

    idx = idx_pad[:, :PPW].reshape(P)
    return (idx, updated)


# speedup vs baseline: 2.9573x; 1.1303x over previous
import functools

import jax
import jax.numpy as jnp
from jax import lax
from jax.experimental import pallas as pl
from jax.experimental.pallas import tpu as pltpu
from jax.experimental.pallas import tpu_sc as plsc

NS, L = 16, 16


def _body(B, P, FLAT, PPW, CH,
          perms_hbm, buf_hbm, scores_hbm, new_hbm, idx_out_hbm, out_hbm,
          cand_f, cand_i, rows_v, gath_v, idxrow_v, idx_all_v, ns_v, bs_v,
          sem):
    wid = lax.axis_index("s")
    p0 = wid * PPW
    pltpu.sync_copy(perms_hbm.at[pl.ds(p0, PPW)], cand_f)
    nk = FLAT // L
    for c in range(PPW):
        for k in range(nk):
            cand_i[c, pl.ds(k * L, L)] = cand_f[c, pl.ds(k * L, L)].astype(jnp.int32)

    lanes = lax.iota(jnp.int32, L)

    def lane_or(x):
        for sh in (8, 4, 2, 1):
            x = x | x[lanes ^ sh]
        return x

    def all_found(iv):
        pending = jnp.where((iv < 0) & (lanes < PPW), 1, 0)
        return lane_or(pending)[0] == 0

    idxrow_v[...] = jnp.full((L,), -1, jnp.int32)

    def chunk_body(chunk, dummy):
        @pl.when(jnp.logical_not(all_found(idxrow_v[...])))
        def _():
            pltpu.async_copy(buf_hbm.at[pl.ds(chunk * CH, CH)], rows_v, sem).wait()

            def row_body(rb, d2):
                iv = idxrow_v[...]

                @pl.when(jnp.logical_not(all_found(iv)))
                def _():
                    accs = [jnp.zeros((L,), jnp.int32) for _ in range(PPW)]
                    for k in range(nk):
                        r = rows_v[rb, pl.ds(k * L, L)].astype(jnp.int32)
                        for c in range(PPW):
                            accs[c] = accs[c] | (r ^ cand_i[c, pl.ds(k * L, L)])
                    b = chunk * CH + rb
                    iv2 = iv
                    for c in range(PPW):
                        match_vec = lane_or(accs[c]) == 0
                        iv2 = jnp.where((lanes == c) & (iv2 < 0) & match_vec, b, iv2)
                    idxrow_v[...] = iv2

                return d2

            lax.fori_loop(0, CH, row_body, jnp.int32(0))

        return dummy

    lax.fori_loop(0, B // CH, chunk_body, jnp.int32(0))

    iv = idxrow_v[...]
    pltpu.async_copy(scores_hbm.at[jnp.maximum(iv, 0)], gath_v, sem).wait()
    g = gath_v[...]
    iv = jnp.where((iv >= 0) & (g != -jnp.inf), iv, -1)
    idxrow_v[...] = iv
    pltpu.sync_copy(idxrow_v, idx_out_hbm.at[wid])

    plsc.subcore_barrier()
    pltpu.sync_copy(idx_out_hbm, idx_all_v)
    pltpu.sync_copy(new_hbm, ns_v)
    rows = B // NS
    base = wid * rows
    pltpu.sync_copy(scores_hbm.at[pl.ds(base, rows)], bs_v)

    nvec = rows // L
    accs = [bs_v[pl.ds(v * L, L)] for v in range(nvec)]
    lanevs = [base + v * L + lanes for v in range(nvec)]
    for t in range(P // L):
        nsvec = ns_v[pl.ds(t * L, L)]
        rowvecs = [idx_all_v[t * (L // PPW) + r] for r in range(L // PPW)]
        for j in range(L):
            ip = rowvecs[j // PPW][j % PPW]
            sp = nsvec[j]
            for v in range(nvec):
                accs[v] = jnp.where(lanevs[v] == ip, jnp.maximum(accs[v], sp), accs[v])
    for v in range(nvec):
        bs_v[pl.ds(v * L, L)] = accs[v]
    pltpu.sync_copy(bs_v, out_hbm.at[pl.ds(base, rows)])


def kernel(perm_buffer, buffer_scores, permutations, new_scores):
    B = perm_buffer.shape[0]
    P = permutations.shape[0]
    FLAT = perm_buffer.shape[1] * perm_buffer.shape[2]
    PPW = P // NS
    CH = 8

    buf2d = perm_buffer.reshape(B, FLAT)
    perm2d = permutations.reshape(P, FLAT)

    mesh = plsc.VectorSubcoreMesh(
        core_axis_name="c", subcore_axis_name="s", num_cores=1)

    run = functools.partial(
        pl.kernel,
        out_type=(
            jax.ShapeDtypeStruct((NS, L), jnp.int32),
            jax.ShapeDtypeStruct((B,), jnp.float32),
        ),
        mesh=mesh,
        scratch_types=[
            pltpu.VMEM((PPW, FLAT), jnp.float32),
            pltpu.VMEM((PPW, FLAT), jnp.int32),
            pltpu.VMEM((CH, FLAT), jnp.float32),
            pltpu.VMEM((L,), jnp.float32),
            pltpu.VMEM((L,), jnp.int32),
            pltpu.VMEM((NS, L), jnp.int32),
            pltpu.VMEM((P,), jnp.float32),
            pltpu.VMEM((B // NS,), jnp.float32),
            pltpu.SemaphoreType.DMA,
        ],
    )(functools.partial(_body, B, P, FLAT, PPW, CH))
    idx_pad, updated = run(perm2d, buf2d, buffer_scores, new_scores)

    idx = idx_pad[:, :PPW].reshape(P)
    return (idx, updated)


# speedup vs baseline: 3.0808x; 1.0418x over previous
import functools

import jax
import jax.numpy as jnp
from jax import lax
from jax.experimental import pallas as pl
from jax.experimental.pallas import tpu as pltpu
from jax.experimental.pallas import tpu_sc as plsc

NS, L = 16, 16


def _body(B, P, FLAT, PPW, CH,
          perms_hbm, buf_hbm, scores_hbm, new_hbm, idx_out_hbm, out_hbm,
          cand_f, cand_i, rows_v, gath_v, idxrow_v, idx_all_v, ns_v, bs_v,
          sem_rows, sem_misc, sem_cand):
    wid = lax.axis_index("s")
    p0 = wid * PPW
    rows = B // NS
    base = wid * rows

    cp_rows0 = pltpu.async_copy(buf_hbm.at[pl.ds(0, CH)], rows_v, sem_rows)
    cp_ns = pltpu.async_copy(new_hbm, ns_v, sem_misc)
    cp_bs = pltpu.async_copy(scores_hbm.at[pl.ds(base, rows)], bs_v, sem_misc)
    cp_cand = pltpu.async_copy(perms_hbm.at[pl.ds(p0, PPW)], cand_f, sem_cand)

    cp_cand.wait()
    nk = FLAT // L
    for c in range(PPW):
        for k in range(nk):
            cand_i[c, pl.ds(k * L, L)] = cand_f[c, pl.ds(k * L, L)].astype(jnp.int32)

    lanes = lax.iota(jnp.int32, L)

    def lane_or(x):
        for sh in (8, 4, 2, 1):
            x = x | x[lanes ^ sh]
        return x

    def all_found(iv):
        pending = jnp.where((iv < 0) & (lanes < PPW), 1, 0)
        return lane_or(pending)[0] == 0

    idxrow_v[...] = jnp.full((L,), -1, jnp.int32)

    def scan_rows(chunk):
        def row_body(rb, d2):
            iv = idxrow_v[...]

            @pl.when(jnp.logical_not(all_found(iv)))
            def _():
                accs = [jnp.zeros((L,), jnp.int32) for _ in range(PPW)]
                for k in range(nk):
                    r = rows_v[rb, pl.ds(k * L, L)].astype(jnp.int32)
                    for c in range(PPW):
                        accs[c] = accs[c] | (r ^ cand_i[c, pl.ds(k * L, L)])
                b = chunk * CH + rb
                iv2 = iv
                for c in range(PPW):
                    match_vec = lane_or(accs[c]) == 0
                    iv2 = jnp.where((lanes == c) & (iv2 < 0) & match_vec, b, iv2)
                idxrow_v[...] = iv2

            return d2

        lax.fori_loop(0, CH, row_body, jnp.int32(0))

    cp_rows0.wait()
    scan_rows(jnp.int32(0))

    def chunk_body(chunk, dummy):
        @pl.when(jnp.logical_not(all_found(idxrow_v[...])))
        def _():
            pltpu.async_copy(
                buf_hbm.at[pl.ds(chunk * CH, CH)], rows_v, sem_rows).wait()
            scan_rows(chunk)

        return dummy

    lax.fori_loop(1, B // CH, chunk_body, jnp.int32(0))

    iv = idxrow_v[...]
    pltpu.async_copy(scores_hbm.at[jnp.maximum(iv, 0)], gath_v, sem_rows).wait()
    g = gath_v[...]
    iv = jnp.where((iv >= 0) & (g != -jnp.inf), iv, -1)
    idxrow_v[...] = iv
    pltpu.sync_copy(idxrow_v.at[pl.ds(0, PPW)], idx_out_hbm.at[pl.ds(p0, PPW)])

    plsc.subcore_barrier()
    pltpu.sync_copy(idx_out_hbm, idx_all_v)
    cp_ns.wait()
    cp_bs.wait()

    nvec = rows // L
    accs = [bs_v[pl.ds(v * L, L)] for v in range(nvec)]
    lanevs = [base + v * L + lanes for v in range(nvec)]
    for t in range(P // L):
        nsvec = ns_v[pl.ds(t * L, L)]
        ivec = idx_all_v[pl.ds(t * L, L)]
        for j in range(L):
            ip = ivec[j]
            sp = nsvec[j]
            for v in range(nvec):
                accs[v] = jnp.where(lanevs[v] == ip, jnp.maximum(accs[v], sp), accs[v])
    for v in range(nvec):
        bs_v[pl.ds(v * L, L)] = accs[v]
    pltpu.sync_copy(bs_v, out_hbm.at[pl.ds(base, rows)])


def kernel(perm_buffer, buffer_scores, permutations, new_scores):
    B = perm_buffer.shape[0]
    P = permutations.shape[0]
    FLAT = perm_buffer.shape[1] * perm_buffer.shape[2]
    PPW = P // NS
    CH = 8

    buf2d = perm_buffer.reshape(B, FLAT)
    perm2d = permutations.reshape(P, FLAT)

    mesh = plsc.VectorSubcoreMesh(
        core_axis_name="c", subcore_axis_name="s", num_cores=1)

    run = functools.partial(
        pl.kernel,
        out_type=(
            jax.ShapeDtypeStruct((P,), jnp.int32),
            jax.ShapeDtypeStruct((B,), jnp.float32),
        ),
        mesh=mesh,
        scratch_types=[
            pltpu.VMEM((PPW, FLAT), jnp.float32),
            pltpu.VMEM((PPW, FLAT), jnp.int32),
            pltpu.VMEM((CH, FLAT), jnp.float32),
            pltpu.VMEM((L,), jnp.float32),
            pltpu.VMEM((L,), jnp.int32),
            pltpu.VMEM((P,), jnp.int32),
            pltpu.VMEM((P,), jnp.float32),
            pltpu.VMEM((B // NS,), jnp.float32),
            pltpu.SemaphoreType.DMA,
            pltpu.SemaphoreType.DMA,
            pltpu.SemaphoreType.DMA,
        ],
    )(functools.partial(_body, B, P, FLAT, PPW, CH))
    idx, updated = run(perm2d, buf2d, buffer_scores, new_scores)
    return (idx, updated)


# speedup vs baseline: 3.5549x; 1.1539x over previous
import functools

import jax
import jax.numpy as jnp
from jax import lax
from jax.experimental import pallas as pl
from jax.experimental.pallas import tpu as pltpu
from jax.experimental.pallas import tpu_sc as plsc

NS, L = 16, 16


def _body(B, P, FLAT, PPW, CH,
          perms_hbm, buf_hbm, scores_hbm, new_hbm, idx_out_hbm, out_hbm,
          cand_f, cand_i, rows_v, gath_v, idxrow_v, idx_all_v, ns_v, bs_v,
          sem_rows, sem_misc, sem_cand):
    wid = lax.axis_index("s")
    p0 = wid * PPW
    rows = B // NS
    base = wid * rows

    cp_rows0 = pltpu.async_copy(buf_hbm.at[pl.ds(0, CH)], rows_v, sem_rows)
    cp_ns = pltpu.async_copy(new_hbm, ns_v, sem_misc)
    cp_bs = pltpu.async_copy(scores_hbm.at[pl.ds(base, rows)], bs_v, sem_misc)
    cp_cand = pltpu.async_copy(perms_hbm.at[pl.ds(p0, PPW)], cand_f, sem_cand)

    cp_cand.wait()
    nk = FLAT // L

    def conv_body(k, d):
        off = k * L
        for c in range(PPW):
            cand_i[c, pl.ds(off, L)] = cand_f[c, pl.ds(off, L)].astype(jnp.int32)
        return d

    lax.fori_loop(0, nk, conv_body, jnp.int32(0))

    lanes = lax.iota(jnp.int32, L)

    def lane_or(x):
        for sh in (8, 4, 2, 1):
            x = x | x[lanes ^ sh]
        return x

    def all_found(iv):
        pending = jnp.where((iv < 0) & (lanes < PPW), 1, 0)
        return lane_or(pending)[0] == 0

    idxrow_v[...] = jnp.full((L,), -1, jnp.int32)

    def scan_rows(chunk):
        def row_body(rb, d2):
            iv = idxrow_v[...]

            @pl.when(jnp.logical_not(all_found(iv)))
            def _():
                def kbody(k, accs):
                    off = k * L
                    r = rows_v[rb, pl.ds(off, L)].astype(jnp.int32)
                    return tuple(
                        accs[c] | (r ^ cand_i[c, pl.ds(off, L)])
                        for c in range(PPW))

                accs = lax.fori_loop(
                    0, nk, kbody,
                    tuple(jnp.zeros((L,), jnp.int32) for _ in range(PPW)))
                b = chunk * CH + rb
                iv2 = iv
                for c in range(PPW):
                    match_vec = lane_or(accs[c]) == 0
                    iv2 = jnp.where((lanes == c) & (iv2 < 0) & match_vec, b, iv2)
                idxrow_v[...] = iv2

            return d2

        lax.fori_loop(0, CH, row_body, jnp.int32(0))

    def chunk_body(chunk, dummy):
        @pl.when(jnp.logical_not(all_found(idxrow_v[...])))
        def _():
            @pl.when(chunk == 0)
            def _():
                cp_rows0.wait()

            @pl.when(chunk > 0)
            def _():
                pltpu.async_copy(
                    buf_hbm.at[pl.ds(chunk * CH, CH)], rows_v, sem_rows).wait()

            scan_rows(chunk)

        return dummy

    lax.fori_loop(0, B // CH, chunk_body, jnp.int32(0))

    iv = idxrow_v[...]
    pltpu.async_copy(scores_hbm.at[jnp.maximum(iv, 0)], gath_v, sem_rows).wait()
    g = gath_v[...]
    iv = jnp.where((iv >= 0) & (g != -jnp.inf), iv, -1)
    idxrow_v[...] = iv
    pltpu.sync_copy(idxrow_v.at[pl.ds(0, PPW)], idx_out_hbm.at[pl.ds(p0, PPW)])

    plsc.subcore_barrier()
    pltpu.sync_copy(idx_out_hbm, idx_all_v)
    cp_ns.wait()
    cp_bs.wait()

    nvec = rows // L
    lanevs = [base + v * L + lanes for v in range(nvec)]

    def tbody(t, accs):
        nsvec = ns_v[pl.ds(t * L, L)]
        ivec = idx_all_v[pl.ds(t * L, L)]
        for j in range(L):
            ip = ivec[j]
            sp = nsvec[j]
            accs = tuple(
                jnp.where(lanevs[v] == ip, jnp.maximum(accs[v], sp), accs[v])
                for v in range(nvec))
        return accs

    accs = lax.fori_loop(
        0, P // L, tbody, tuple(bs_v[pl.ds(v * L, L)] for v in range(nvec)))
    for v in range(nvec):
        bs_v[pl.ds(v * L, L)] = accs[v]
    pltpu.sync_copy(bs_v, out_hbm.at[pl.ds(base, rows)])


def kernel(perm_buffer, buffer_scores, permutations, new_scores):
    B = perm_buffer.shape[0]
    P = permutations.shape[0]
    FLAT = perm_buffer.shape[1] * perm_buffer.shape[2]
    PPW = P // NS
    CH = 8

    buf2d = perm_buffer.reshape(B, FLAT)
    perm2d = permutations.reshape(P, FLAT)

    mesh = plsc.VectorSubcoreMesh(
        core_axis_name="c", subcore_axis_name="s", num_cores=1)

    run = functools.partial(
        pl.kernel,
        out_type=(
            jax.ShapeDtypeStruct((P,), jnp.int32),
            jax.ShapeDtypeStruct((B,), jnp.float32),
        ),
        mesh=mesh,
        scratch_types=[
            pltpu.VMEM((PPW, FLAT), jnp.float32),
            pltpu.VMEM((PPW, FLAT), jnp.int32),
            pltpu.VMEM((CH, FLAT), jnp.float32),
            pltpu.VMEM((L,), jnp.float32),
            pltpu.VMEM((L,), jnp.int32),
            pltpu.VMEM((P,), jnp.int32),
            pltpu.VMEM((P,), jnp.float32),
            pltpu.VMEM((B // NS,), jnp.float32),
            pltpu.SemaphoreType.DMA,
            pltpu.SemaphoreType.DMA,
            pltpu.SemaphoreType.DMA,
        ],
    )(functools.partial(_body, B, P, FLAT, PPW, CH))
    idx, updated = run(perm2d, buf2d, buffer_scores, new_scores)
    return (idx, updated)


# speedup vs baseline: 3.6930x; 1.0388x over previous
import functools

import jax
import jax.numpy as jnp
from jax import lax
from jax.experimental import pallas as pl
from jax.experimental.pallas import tpu as pltpu
from jax.experimental.pallas import tpu_sc as plsc

NS, L = 16, 16


def _body(B, P, FLAT, PPW, CH,
          perms_hbm, buf_hbm, scores_hbm, new_hbm, idx_out_hbm, out_hbm,
          cand_f, rows_v, scores_v, idxrow_v, idx_all_v, ns_v, bs_v,
          sem_rows, sem_misc, sem_cand):
    wid = lax.axis_index("s")
    p0 = wid * PPW
    rows = B // NS
    base = wid * rows

    cp_rows0 = pltpu.async_copy(buf_hbm.at[pl.ds(0, CH)], rows_v, sem_rows)
    cp_ns = pltpu.async_copy(new_hbm, ns_v, sem_misc)
    cp_bs = pltpu.async_copy(scores_hbm.at[pl.ds(base, rows)], bs_v, sem_misc)
    cp_sc = pltpu.async_copy(scores_hbm, scores_v, sem_misc)
    cp_cand = pltpu.async_copy(perms_hbm.at[pl.ds(p0, PPW)], cand_f, sem_cand)

    cp_cand.wait()
    nk = FLAT // L

    lanes = lax.iota(jnp.int32, L)

    def lane_or(x):
        for sh in (8, 4, 2, 1):
            x = x | x[lanes ^ sh]
        return x

    def all_found(iv):
        pending = jnp.where((iv < 0) & (lanes < PPW), 1, 0)
        return lane_or(pending)[0] == 0

    idxrow_v[...] = jnp.full((L,), -1, jnp.int32)

    def scan_rows(chunk):
        def row_body(rb, d2):
            iv = idxrow_v[...]

            @pl.when(jnp.logical_not(all_found(iv)))
            def _():
                def kbody(k, accs):
                    off = k * L
                    r = rows_v[rb, pl.ds(off, L)].astype(jnp.int32)
                    return tuple(
                        accs[c] | (r ^ cand_f[c, pl.ds(off, L)].astype(jnp.int32))
                        for c in range(PPW))

                accs = lax.fori_loop(
                    0, nk, kbody,
                    tuple(jnp.zeros((L,), jnp.int32) for _ in range(PPW)))
                b = chunk * CH + rb
                iv2 = iv
                for c in range(PPW):
                    match_vec = lane_or(accs[c]) == 0
                    iv2 = jnp.where((lanes == c) & (iv2 < 0) & match_vec, b, iv2)
                idxrow_v[...] = iv2

            return d2

        lax.fori_loop(0, CH, row_body, jnp.int32(0))

    def chunk_body(chunk, dummy):
        @pl.when(jnp.logical_not(all_found(idxrow_v[...])))
        def _():
            @pl.when(chunk == 0)
            def _():
                cp_rows0.wait()

            @pl.when(chunk > 0)
            def _():
                pltpu.async_copy(
                    buf_hbm.at[pl.ds(chunk * CH, CH)], rows_v, sem_rows).wait()

            scan_rows(chunk)

        return dummy

    lax.fori_loop(0, B // CH, chunk_body, jnp.int32(0))

    iv = idxrow_v[...]
    cp_sc.wait()

    def gbody(ch, g):
        svec = scores_v[pl.ds(ch * L, L)]
        val = svec[iv & (L - 1)]
        return jnp.where((iv >> 4) == ch, val, g)

    g = lax.fori_loop(0, B // L, gbody, jnp.full((L,), -jnp.inf, jnp.float32))
    iv = jnp.where((iv >= 0) & (g != -jnp.inf), iv, -1)
    idxrow_v[...] = iv
    pltpu.sync_copy(idxrow_v.at[pl.ds(0, PPW)], idx_out_hbm.at[pl.ds(p0, PPW)])

    plsc.subcore_barrier()
    pltpu.sync_copy(idx_out_hbm, idx_all_v)
    cp_ns.wait()
    cp_bs.wait()

    nvec = rows // L
    lanevs = [base + v * L + lanes for v in range(nvec)]

    def tbody(t, accs):
        nsvec = ns_v[pl.ds(t * L, L)]
        ivec = idx_all_v[pl.ds(t * L, L)]
        for j in range(L):
            ip = ivec[j]
            sp = nsvec[j]
            accs = tuple(
                jnp.where(lanevs[v] == ip, jnp.maximum(accs[v], sp), accs[v])
                for v in range(nvec))
        return accs

    accs = lax.fori_loop(
        0, P // L, tbody, tuple(bs_v[pl.ds(v * L, L)] for v in range(nvec)))
    for v in range(nvec):
        bs_v[pl.ds(v * L, L)] = accs[v]
    pltpu.sync_copy(bs_v, out_hbm.at[pl.ds(base, rows)])


def kernel(perm_buffer, buffer_scores, permutations, new_scores):
    B = perm_buffer.shape[0]
    P = permutations.shape[0]
    FLAT = perm_buffer.shape[1] * perm_buffer.shape[2]
    PPW = P // NS
    CH = 8

    buf2d = perm_buffer.reshape(B, FLAT)
    perm2d = permutations.reshape(P, FLAT)

    mesh = plsc.VectorSubcoreMesh(
        core_axis_name="c", subcore_axis_name="s", num_cores=1)

    run = functools.partial(
        pl.kernel,
        out_type=(
            jax.ShapeDtypeStruct((P,), jnp.int32),
            jax.ShapeDtypeStruct((B,), jnp.float32),
        ),
        mesh=mesh,
        scratch_types=[
            pltpu.VMEM((PPW, FLAT), jnp.float32),
            pltpu.VMEM((CH, FLAT), jnp.float32),
            pltpu.VMEM((B,), jnp.float32),
            pltpu.VMEM((L,), jnp.int32),
            pltpu.VMEM((P,), jnp.int32),
            pltpu.VMEM((P,), jnp.float32),
            pltpu.VMEM((B // NS,), jnp.float32),
            pltpu.SemaphoreType.DMA,
            pltpu.SemaphoreType.DMA,
            pltpu.SemaphoreType.DMA,
        ],
    )(functools.partial(_body, B, P, FLAT, PPW, CH))
    idx, updated = run(perm2d, buf2d, buffer_scores, new_scores)
    return (idx, updated)


# speedup vs baseline: 3.9219x; 1.0620x over previous
import functools

import jax
import jax.numpy as jnp
from jax import lax
from jax.experimental import pallas as pl
from jax.experimental.pallas import tpu as pltpu
from jax.experimental.pallas import tpu_sc as plsc

NS, L = 16, 16


def _body(B, P, FLAT, PPW, CH,
          perms_hbm, buf_hbm, scores_hbm, new_hbm, idx_out_hbm, out_hbm,
          cand_f, rows_v, scores_v, idxrow_v, idx_all_v, ns_v, bs_v,
          sem_rows, sem_misc, sem_cand):
    wid = lax.axis_index("s")
    p0 = wid * PPW
    rows = B // NS
    base = wid * rows

    cp_rows0 = pltpu.async_copy(buf_hbm.at[pl.ds(0, CH)], rows_v, sem_rows)
    cp_ns = pltpu.async_copy(new_hbm, ns_v, sem_misc)
    cp_bs = pltpu.async_copy(scores_hbm.at[pl.ds(base, rows)], bs_v, sem_misc)
    cp_sc = pltpu.async_copy(scores_hbm, scores_v, sem_misc)
    cp_cand = pltpu.async_copy(perms_hbm.at[pl.ds(p0, PPW)], cand_f, sem_cand)

    cp_cand.wait()
    nk = FLAT // L

    lanes = lax.iota(jnp.int32, L)

    def lane_or(x):
        for sh in (8, 4, 2, 1):
            x = x | x[lanes ^ sh]
        return x

    def all_found(iv):
        pending = jnp.where((iv < 0) & (lanes < PPW), 1, 0)
        return lane_or(pending)[0] == 0

    idxrow_v[...] = jnp.full((L,), -1, jnp.int32)

    def scan_rows(chunk):
        def row_body(rb, d2):
            iv = idxrow_v[...]

            @pl.when(jnp.logical_not(all_found(iv)))
            def _():
                KU = 4

                def kbody(k, accs):
                    accs = list(accs)
                    for u in range(KU):
                        off = (k * KU + u) * L
                        r = rows_v[rb, pl.ds(off, L)].astype(jnp.int32)
                        for c in range(PPW):
                            accs[c] = accs[c] | (
                                r ^ cand_f[c, pl.ds(off, L)].astype(jnp.int32))
                    return tuple(accs)

                accs = lax.fori_loop(
                    0, nk // KU, kbody,
                    tuple(jnp.zeros((L,), jnp.int32) for _ in range(PPW)))
                b = chunk * CH + rb
                iv2 = iv
                for c in range(PPW):
                    match_vec = lane_or(accs[c]) == 0
                    iv2 = jnp.where((lanes == c) & (iv2 < 0) & match_vec, b, iv2)
                idxrow_v[...] = iv2

            return d2

        lax.fori_loop(0, CH, row_body, jnp.int32(0))

    cp_rows0.wait()
    scan_rows(jnp.int32(0))

    @pl.when(jnp.logical_not(all_found(idxrow_v[...])))
    def _():
        def chunk_body(chunk, dummy):
            @pl.when(jnp.logical_not(all_found(idxrow_v[...])))
            def _():
                pltpu.async_copy(
                    buf_hbm.at[pl.ds(chunk * CH, CH)], rows_v, sem_rows).wait()
                scan_rows(chunk)

            return dummy

        lax.fori_loop(1, B // CH, chunk_body, jnp.int32(0))

    iv = idxrow_v[...]
    cp_sc.wait()

    def gbody(ch, g):
        svec = scores_v[pl.ds(ch * L, L)]
        val = svec[iv & (L - 1)]
        return jnp.where((iv >> 4) == ch, val, g)

    g = lax.fori_loop(0, B // L, gbody, jnp.full((L,), -jnp.inf, jnp.float32))
    iv = jnp.where((iv >= 0) & (g != -jnp.inf), iv, -1)
    idxrow_v[...] = iv
    pltpu.sync_copy(idxrow_v.at[pl.ds(0, PPW)], idx_out_hbm.at[pl.ds(p0, PPW)])

    plsc.subcore_barrier()
    pltpu.sync_copy(idx_out_hbm, idx_all_v)
    cp_ns.wait()
    cp_bs.wait()

    nvec = rows // L
    lanevs = [base + v * L + lanes for v in range(nvec)]

    def tbody(t, accs):
        nsvec = ns_v[pl.ds(t * L, L)]
        ivec = idx_all_v[pl.ds(t * L, L)]
        for j in range(L):
            ip = ivec[j]
            sp = nsvec[j]
            accs = tuple(
                jnp.where(lanevs[v] == ip, jnp.maximum(accs[v], sp), accs[v])
                for v in range(nvec))
        return accs

    accs = lax.fori_loop(
        0, P // L, tbody, tuple(bs_v[pl.ds(v * L, L)] for v in range(nvec)))
    for v in range(nvec):
        bs_v[pl.ds(v * L, L)] = accs[v]
    pltpu.sync_copy(bs_v, out_hbm.at[pl.ds(base, rows)])


def kernel(perm_buffer, buffer_scores, permutations, new_scores):
    B = perm_buffer.shape[0]
    P = permutations.shape[0]
    FLAT = perm_buffer.shape[1] * perm_buffer.shape[2]
    PPW = P // NS
    CH = 4

    buf2d = perm_buffer.reshape(B, FLAT)
    perm2d = permutations.reshape(P, FLAT)

    mesh = plsc.VectorSubcoreMesh(
        core_axis_name="c", subcore_axis_name="s", num_cores=1)

    run = functools.partial(
        pl.kernel,
        out_type=(
            jax.ShapeDtypeStruct((P,), jnp.int32),
            jax.ShapeDtypeStruct((B,), jnp.float32),
        ),
        mesh=mesh,
        scratch_types=[
            pltpu.VMEM((PPW, FLAT), jnp.float32),
            pltpu.VMEM((CH, FLAT), jnp.float32),
            pltpu.VMEM((B,), jnp.float32),
            pltpu.VMEM((L,), jnp.int32),
            pltpu.VMEM((P,), jnp.int32),
            pltpu.VMEM((P,), jnp.float32),
            pltpu.VMEM((B // NS,), jnp.float32),
            pltpu.SemaphoreType.DMA,
            pltpu.SemaphoreType.DMA,
            pltpu.SemaphoreType.DMA,
        ],
    )(functools.partial(_body, B, P, FLAT, PPW, CH))
    idx, updated = run(perm2d, buf2d, buffer_scores, new_scores)
    return (idx, updated)


# speedup vs baseline: 4.0791x; 1.0401x over previous
import functools

import jax
import jax.numpy as jnp
from jax import lax
from jax.experimental import pallas as pl
from jax.experimental.pallas import tpu as pltpu
from jax.experimental.pallas import tpu_sc as plsc

NS, L = 16, 16


def _body(B, P, FLAT, PPW, CH,
          perms_hbm, buf_hbm, scores_hbm, new_hbm, idx_out_hbm, out_hbm,
          cand_f, rows_v, scores_v, idxrow_v, idx_all_v, ns_v, bs_v,
          shared_idx, sem_rows, sem_misc, sem_cand):
    wid = lax.axis_index("s")
    p0 = wid * PPW
    rows = B // NS
    base = wid * rows

    cp_rows0 = pltpu.async_copy(buf_hbm.at[pl.ds(0, CH)], rows_v, sem_rows)
    cp_ns = pltpu.async_copy(new_hbm, ns_v, sem_misc)
    cp_bs = pltpu.async_copy(scores_hbm.at[pl.ds(base, rows)], bs_v, sem_misc)
    cp_sc = pltpu.async_copy(scores_hbm, scores_v, sem_misc)
    cp_cand = pltpu.async_copy(perms_hbm.at[pl.ds(p0, PPW)], cand_f, sem_cand)

    cp_cand.wait()
    nk = FLAT // L

    lanes = lax.iota(jnp.int32, L)

    def lane_or(x):
        for sh in (8, 4, 2, 1):
            x = x | x[lanes ^ sh]
        return x

    def all_found(iv):
        pending = jnp.where((iv < 0) & (lanes < PPW), 1, 0)
        return lane_or(pending)[0] == 0

    idxrow_v[...] = jnp.full((L,), -1, jnp.int32)

    def scan_rows(chunk):
        def row_body(rb, d2):
            iv = idxrow_v[...]

            @pl.when(jnp.logical_not(all_found(iv)))
            def _():
                KU = 4

                def kbody(k, accs):
                    accs = list(accs)
                    for u in range(KU):
                        off = (k * KU + u) * L
                        r = rows_v[rb, pl.ds(off, L)].astype(jnp.int32)
                        for c in range(PPW):
                            accs[c] = accs[c] | (
                                r ^ cand_f[c, pl.ds(off, L)].astype(jnp.int32))
                    return tuple(accs)

                accs = lax.fori_loop(
                    0, nk // KU, kbody,
                    tuple(jnp.zeros((L,), jnp.int32) for _ in range(PPW)))
                b = chunk * CH + rb
                iv2 = iv
                for c in range(PPW):
                    match_vec = lane_or(accs[c]) == 0
                    iv2 = jnp.where((lanes == c) & (iv2 < 0) & match_vec, b, iv2)
                idxrow_v[...] = iv2

            return d2

        lax.fori_loop(0, CH, row_body, jnp.int32(0))

    cp_rows0.wait()
    scan_rows(jnp.int32(0))

    @pl.when(jnp.logical_not(all_found(idxrow_v[...])))
    def _():
        def chunk_body(chunk, dummy):
            @pl.when(jnp.logical_not(all_found(idxrow_v[...])))
            def _():
                pltpu.async_copy(
                    buf_hbm.at[pl.ds(chunk * CH, CH)], rows_v, sem_rows).wait()
                scan_rows(chunk)

            return dummy

        lax.fori_loop(1, B // CH, chunk_body, jnp.int32(0))

    iv = idxrow_v[...]
    cp_sc.wait()

    def gbody(ch, g):
        svec = scores_v[pl.ds(ch * L, L)]
        val = svec[iv & (L - 1)]
        return jnp.where((iv >> 4) == ch, val, g)

    g = lax.fori_loop(0, B // L, gbody, jnp.full((L,), -jnp.inf, jnp.float32))
    iv = jnp.where((iv >= 0) & (g != -jnp.inf), iv, -1)
    idxrow_v[...] = iv
    pltpu.sync_copy(idxrow_v.at[pl.ds(0, PPW)], idx_out_hbm.at[pl.ds(p0, PPW)])

    pltpu.sync_copy(idxrow_v.at[pl.ds(0, PPW)], shared_idx.at[pl.ds(p0, PPW)])
    plsc.subcore_barrier()
    pltpu.sync_copy(shared_idx, idx_all_v)
    cp_ns.wait()
    cp_bs.wait()

    nvec = rows // L
    lanevs = [base + v * L + lanes for v in range(nvec)]

    def tbody(t, accs):
        nsvec = ns_v[pl.ds(t * L, L)]
        ivec = idx_all_v[pl.ds(t * L, L)]
        for j in range(L):
            ip = ivec[j]
            sp = nsvec[j]
            accs = tuple(
                jnp.where(lanevs[v] == ip, jnp.maximum(accs[v], sp), accs[v])
                for v in range(nvec))
        return accs

    accs = lax.fori_loop(
        0, P // L, tbody, tuple(bs_v[pl.ds(v * L, L)] for v in range(nvec)))
    for v in range(nvec):
        bs_v[pl.ds(v * L, L)] = accs[v]
    pltpu.sync_copy(bs_v, out_hbm.at[pl.ds(base, rows)])


def kernel(perm_buffer, buffer_scores, permutations, new_scores):
    B = perm_buffer.shape[0]
    P = permutations.shape[0]
    FLAT = perm_buffer.shape[1] * perm_buffer.shape[2]
    PPW = P // NS
    CH = 4

    buf2d = perm_buffer.reshape(B, FLAT)
    perm2d = permutations.reshape(P, FLAT)

    mesh = plsc.VectorSubcoreMesh(
        core_axis_name="c", subcore_axis_name="s", num_cores=1)

    run = functools.partial(
        pl.kernel,
        out_type=(
            jax.ShapeDtypeStruct((P,), jnp.int32),
            jax.ShapeDtypeStruct((B,), jnp.float32),
        ),
        mesh=mesh,
        scratch_types=[
            pltpu.VMEM((PPW, FLAT), jnp.float32),
            pltpu.VMEM((CH, FLAT), jnp.float32),
            pltpu.VMEM((B,), jnp.float32),
            pltpu.VMEM((L,), jnp.int32),
            pltpu.VMEM((P,), jnp.int32),
            pltpu.VMEM((P,), jnp.float32),
            pltpu.VMEM((B // NS,), jnp.float32),
            pltpu.MemorySpace.VMEM_SHARED((P,), jnp.int32),
            pltpu.SemaphoreType.DMA,
            pltpu.SemaphoreType.DMA,
            pltpu.SemaphoreType.DMA,
        ],
    )(functools.partial(_body, B, P, FLAT, PPW, CH))
    idx, updated = run(perm2d, buf2d, buffer_scores, new_scores)
    return (idx, updated)
